# Initial kernel scaffold; baseline (speedup 1.0000x reference)
import jax
import jax.numpy as jnp
from jax.experimental import pallas as pl


def kernel(tokens, embed_table, W, b):
    raise NotImplementedError("write your pallas kernel here")


# speedup vs baseline: 5.1795x; 5.1795x over previous
import functools

import jax
import jax.numpy as jnp
from jax.experimental import pallas as pl

B, L, D, V = 1024, 512, 640, 5
VPAD = 8
BB = 8


def _ptable_body(et_ref, w_ref, b_ref, out_ref):
    et = et_ref[...]
    w = w_ref[...]
    proj = jax.lax.dot_general(
        et, w, (((1,), (1,)), ((), ())), preferred_element_type=jnp.float32
    )
    out_ref[...] = proj + b_ref[...][None, :]


def _lookup_body(tok_ref, pt_ref, out_ref, pooled_ref):
    t = tok_ref[...]
    pt = pt_ref[...]
    emb = jnp.zeros((BB, L, D), jnp.float32)
    pooled = jnp.zeros((BB, D), jnp.float32)
    for v in range(V):
        mf = (t == v).astype(jnp.float32)
        row = pt[v][None, :]
        emb = emb + mf[:, :, None] * row[None]
        cnt = jnp.sum(mf, axis=1)
        pooled = pooled + cnt[:, None] * row
    out_ref[...] = emb
    pooled_ref[...] = pooled * (1.0 / L)


@jax.jit
def kernel(tokens, embed_table, W, b):
    et_pad = jnp.zeros((VPAD, D), jnp.float32).at[:V].set(embed_table)
    ptable = pl.pallas_call(
        _ptable_body,
        out_shape=jax.ShapeDtypeStruct((VPAD, D), jnp.float32),
    )(et_pad, W, b)

    grid = (B // BB,)
    emb, pooled = pl.pallas_call(
        _lookup_body,
        grid=grid,
        in_specs=[
            pl.BlockSpec((BB, L), lambda i: (i, 0)),
            pl.BlockSpec((VPAD, D), lambda i: (0, 0)),
        ],
        out_specs=[
            pl.BlockSpec((BB, L, D), lambda i: (i, 0, 0)),
            pl.BlockSpec((BB, D), lambda i: (i, 0)),
        ],
        out_shape=[
            jax.ShapeDtypeStruct((B, L, D), jnp.float32),
            jax.ShapeDtypeStruct((B, D), jnp.float32),
        ],
    )(tokens, ptable)
    return (emb, pooled, tokens)


# speedup vs baseline: 5.5067x; 1.0632x over previous
import functools

import jax
import jax.numpy as jnp
from jax import lax
from jax.experimental import pallas as pl
from jax.experimental.pallas import tpu as pltpu
from jax.experimental.pallas import tpu_sc as plsc

B, L, D, V = 1024, 512, 640, 5
VPAD = 8
K2 = 16
BB = 8
NSB = B // BB

NC, NS = 2, 16
NW = NC * NS
SB_W = NSB // NW
TOK_W = SB_W * L * BB
CNT_SB = VPAD * 16
CNT_W = SB_W * CNT_SB


def _ptable_body(et_ref, w_ref, b_ref, pt_ref, t2_ref):
    et = et_ref[...]
    w = w_ref[...]
    proj = lax.dot_general(
        et, w, (((1,), (1,)), ((), ())), preferred_element_type=jnp.float32
    )
    pt = proj + b_ref[...][None, :]
    hi = pt.astype(jnp.bfloat16)
    lo = (pt - hi.astype(jnp.float32)).astype(jnp.bfloat16)
    pt_ref[...] = pt
    t2_ref[...] = jnp.concatenate([hi, lo], axis=0)


def _lookup_body(tokT_ref, t2_ref, out_ref):
    tb = tokT_ref[0]
    t2 = t2_ref[...]
    iota = lax.broadcasted_iota(jnp.int32, (1, K2), 1) & 7
    for bb in range(BB):
        col = tb[:, bb : bb + 1]
        oh = (col == iota)
        emb_bb = lax.dot_general(
            oh.astype(jnp.bfloat16), t2,
            (((1,), (0,)), ((), ())), preferred_element_type=jnp.float32,
        )
        out_ref[bb] = emb_bb


def _hist_sc_body(tok_hbm, cnt_hbm, tok_v, cnt_v):
    wid = lax.axis_index("s") * NC + lax.axis_index("c")
    pltpu.sync_copy(tok_hbm.at[pl.ds(wid * TOK_W, TOK_W)], tok_v)
    for s in range(SB_W):
        def step(i, accs):
            tv = tok_v[pl.ds(s * (L * BB) + i * 16, 16)]
            return tuple(
                a + jnp.where(tv == v, 1.0, 0.0)
                for v, a in enumerate(accs)
            )

        accs = lax.fori_loop(
            0, L * BB // 16, step,
            tuple(jnp.zeros((16,), jnp.float32) for _ in range(V)),
        )
        for v in range(V):
            cnt_v[pl.ds(s * CNT_SB + v * 16, 16)] = accs[v]
        for v in range(V, VPAD):
            cnt_v[pl.ds(s * CNT_SB + v * 16, 16)] = jnp.zeros(
                (16,), jnp.float32)
    pltpu.sync_copy(cnt_v, cnt_hbm.at[pl.ds(wid * CNT_W, CNT_W)])


_hist_sc = functools.partial(
    pl.kernel,
    mesh=plsc.VectorSubcoreMesh(core_axis_name="c", subcore_axis_name="s"),
    out_type=jax.ShapeDtypeStruct((NSB * CNT_SB,), jnp.float32),
    scratch_types=[
        pltpu.VMEM((TOK_W,), jnp.int32),
        pltpu.VMEM((CNT_W,), jnp.float32),
    ],
)(_hist_sc_body)


def _pooled_body(cnt_ref, pt_ref, out_ref):
    cp = cnt_ref[0]
    cpf = cp[:, :8] + cp[:, 8:]
    pooled = lax.dot_general(
        cpf, pt_ref[...], (((0,), (0,)), ((), ())),
        precision=lax.Precision.HIGHEST,
        preferred_element_type=jnp.float32,
    )
    out_ref[...] = pooled * (1.0 / L)


@jax.jit
def kernel(tokens, embed_table, W, b):
    et_pad = jnp.zeros((VPAD, D), jnp.float32).at[:V].set(embed_table)
    ptable, table2 = pl.pallas_call(
        _ptable_body,
        out_shape=[
            jax.ShapeDtypeStruct((VPAD, D), jnp.float32),
            jax.ShapeDtypeStruct((K2, D), jnp.bfloat16),
        ],
    )(et_pad, W, b)

    tokens_T = tokens.reshape(NSB, BB, L).transpose(0, 2, 1)

    counts = _hist_sc(tokens_T.reshape(-1)).reshape(NSB, VPAD, 16)

    emb = pl.pallas_call(
        _lookup_body,
        grid=(NSB,),
        in_specs=[
            pl.BlockSpec((1, L, BB), lambda i: (i, 0, 0)),
            pl.BlockSpec((K2, D), lambda i: (0, 0)),
        ],
        out_specs=pl.BlockSpec((BB, L, D), lambda i: (i, 0, 0)),
        out_shape=jax.ShapeDtypeStruct((B, L, D), jnp.float32),
    )(tokens_T, table2)

    pooled = pl.pallas_call(
        _pooled_body,
        grid=(NSB,),
        in_specs=[
            pl.BlockSpec((1, VPAD, 16), lambda i: (i, 0, 0)),
            pl.BlockSpec((VPAD, D), lambda i: (0, 0)),
        ],
        out_specs=pl.BlockSpec((BB, D), lambda i: (i, 0)),
        out_shape=jax.ShapeDtypeStruct((B, D), jnp.float32),
    )(counts, ptable)
    return (emb, pooled, tokens)


# speedup vs baseline: 6.2631x; 1.1374x over previous
import functools

import jax
import jax.numpy as jnp
from jax import lax
from jax.experimental import pallas as pl
from jax.experimental.pallas import tpu as pltpu
from jax.experimental.pallas import tpu_sc as plsc

B, L, D, V = 1024, 512, 640, 5
VPAD = 8
K2 = 16
BB = 8
NSB = B // BB

NC, NS = 2, 16
NW = NC * NS
SB_W = NSB // NW
TOK_W = SB_W * L * BB
CNT_SB = VPAD * 16
CNT_W = SB_W * CNT_SB


def _ptable_body(et_ref, w_ref, b_ref, pt_ref, t2_ref):
    et = et_ref[...]
    w = w_ref[...]
    proj = lax.dot_general(
        et, w, (((1,), (1,)), ((), ())), preferred_element_type=jnp.float32
    )
    pt = proj + b_ref[...][None, :]
    hi = pt.astype(jnp.bfloat16)
    lo = (pt - hi.astype(jnp.float32)).astype(jnp.bfloat16)
    pt_ref[...] = pt
    t2_ref[...] = jnp.concatenate([hi, lo], axis=0)


def _lookup_body(tokT_ref, t2_ref, out_ref):
    tb = tokT_ref[0]
    t2 = t2_ref[...]
    iota = lax.broadcasted_iota(jnp.int32, (1, K2), 1) & 7
    for bb in range(BB):
        col = tb[:, bb : bb + 1]
        oh = (col == iota)
        emb_bb = lax.dot_general(
            oh.astype(jnp.bfloat16), t2,
            (((1,), (0,)), ((), ())), preferred_element_type=jnp.float32,
        )
        out_ref[bb] = emb_bb


def _hist_sc_body(tok_hbm, cnt_hbm, tok_v, cnt_v):
    wid = lax.axis_index("s") * NC + lax.axis_index("c")
    pltpu.sync_copy(tok_hbm.at[pl.ds(wid * TOK_W, TOK_W)], tok_v)
    for s in range(SB_W):
        def step(i, accs):
            tv = tok_v[pl.ds(s * (L * BB) + i * 16, 16)]
            return tuple(
                a + jnp.where(tv == v, 1.0, 0.0)
                for v, a in enumerate(accs)
            )

        accs = lax.fori_loop(
            0, L * BB // 16, step,
            tuple(jnp.zeros((16,), jnp.float32) for _ in range(V)),
        )
        for v in range(V):
            cnt_v[pl.ds(s * CNT_SB + v * 16, 16)] = accs[v]
        for v in range(V, VPAD):
            cnt_v[pl.ds(s * CNT_SB + v * 16, 16)] = jnp.zeros(
                (16,), jnp.float32)
    pltpu.sync_copy(cnt_v, cnt_hbm.at[pl.ds(wid * CNT_W, CNT_W)])


_hist_sc = functools.partial(
    pl.kernel,
    mesh=plsc.VectorSubcoreMesh(core_axis_name="c", subcore_axis_name="s"),
    out_type=jax.ShapeDtypeStruct((NSB * CNT_SB,), jnp.float32),
    scratch_types=[
        pltpu.VMEM((TOK_W,), jnp.int32),
        pltpu.VMEM((CNT_W,), jnp.float32),
    ],
)(_hist_sc_body)


def _pooled_body(cnt_ref, pt_ref, out_ref):
    cp = cnt_ref[...]
    cpf = cp[:, :, :8] + cp[:, :, 8:]
    cmat = cpf.transpose(0, 2, 1).reshape(B, VPAD)
    pooled = lax.dot_general(
        cmat, pt_ref[...], (((1,), (0,)), ((), ())),
        precision=lax.Precision.HIGHEST,
        preferred_element_type=jnp.float32,
    )
    out_ref[...] = pooled * (1.0 / L)


@jax.jit
def kernel(tokens, embed_table, W, b):
    et_pad = jnp.zeros((VPAD, D), jnp.float32).at[:V].set(embed_table)
    ptable, table2 = pl.pallas_call(
        _ptable_body,
        out_shape=[
            jax.ShapeDtypeStruct((VPAD, D), jnp.float32),
            jax.ShapeDtypeStruct((K2, D), jnp.bfloat16),
        ],
    )(et_pad, W, b)

    tokens_T = tokens.reshape(NSB, BB, L).transpose(0, 2, 1)

    counts = _hist_sc(tokens_T.reshape(-1)).reshape(NSB, VPAD, 16)

    emb = pl.pallas_call(
        _lookup_body,
        grid=(NSB,),
        in_specs=[
            pl.BlockSpec((1, L, BB), lambda i: (i, 0, 0)),
            pl.BlockSpec((K2, D), lambda i: (0, 0)),
        ],
        out_specs=pl.BlockSpec((BB, L, D), lambda i: (i, 0, 0)),
        out_shape=jax.ShapeDtypeStruct((B, L, D), jnp.float32),
    )(tokens_T, table2)

    pooled = pl.pallas_call(
        _pooled_body,
        out_shape=jax.ShapeDtypeStruct((B, D), jnp.float32),
    )(counts, ptable)
    return (emb, pooled, tokens)
